# baseline (speedup 1.0000x reference)
import functools

import jax
import jax.numpy as jnp
from jax import lax
from jax.experimental import pallas as pl
from jax.experimental.pallas import tpu as pltpu
from jax.experimental.pallas import tpu_sc as plsc

_ROWS = 128
_COLS = 32768
_NC = 2
_NS = 16
_NW = _NC * _NS
_RPW = _ROWS // _NW
_L = 16
_UNROLL = 8
_NIT = _COLS // (_L * _UNROLL)
_BIG = jnp.int32(1 << 30)


def _permute(x, perm):
    dn = lax.GatherDimensionNumbers(
        offset_dims=(), collapsed_slice_dims=(0,), start_index_map=(0,)
    )
    return lax.gather(
        x, perm[:, None], dn, slice_sizes=(1,),
        mode=lax.GatherScatterMode.PROMISE_IN_BOUNDS,
    )


def _merge(m, e, pm, pe):
    better = (pm > m) | ((pm == m) & (pe < e))
    return jnp.where(better, pm, m), jnp.where(better, pe, e)


def _row_argmax(buf):
    iota = lax.iota(jnp.int32, _L)
    neg = jnp.full((_L,), -jnp.inf, dtype=jnp.float32)
    zero = jnp.zeros((_L,), dtype=jnp.int32)

    def step(it, carry):
        ms = carry[:_UNROLL]
        js = carry[_UNROLL:]
        new_ms, new_js = [], []
        base = it * _UNROLL
        for a in range(_UNROLL):
            j = base + a
            v = buf[pl.ds(j * _L, _L)]
            gt = v > ms[a]
            new_ms.append(jnp.where(gt, v, ms[a]))
            new_js.append(jnp.where(gt, j, js[a]))
        return tuple(new_ms) + tuple(new_js)

    carry = lax.fori_loop(0, _NIT, step, (neg,) * _UNROLL + (zero,) * _UNROLL)
    ms = carry[:_UNROLL]
    js = carry[_UNROLL:]

    m, e = ms[0], js[0] * _L + iota
    for a in range(1, _UNROLL):
        m, e = _merge(m, e, ms[a], js[a] * _L + iota)
    for s in (8, 4, 2, 1):
        perm = iota ^ s
        pm = _permute(m, perm)
        pe = _permute(e, perm)
        m, e = _merge(m, e, pm, pe)
    return e


@functools.partial(
    pl.kernel,
    out_type=jax.ShapeDtypeStruct((_NW, _L), jnp.int32),
    mesh=plsc.VectorSubcoreMesh(core_axis_name="c", subcore_axis_name="s"),
    scratch_types=[
        pltpu.VMEM((_COLS,), jnp.float32),
        pltpu.VMEM((_COLS,), jnp.float32),
        pltpu.VMEM((_L,), jnp.int32),
        pltpu.SemaphoreType.DMA,
        pltpu.SemaphoreType.DMA,
    ],
)
def _argmax_sc(x_hbm, out_hbm, buf0, buf1, outv, sem0, sem1):
    wid = lax.axis_index("s") * _NC + lax.axis_index("c")
    row0 = wid * _RPW
    bufs = (buf0, buf1)
    sems = (sem0, sem1)
    iota = lax.iota(jnp.int32, _L)
    copies = [pltpu.async_copy(x_hbm.at[row0], buf0, sem0)]
    acc = jnp.zeros((_L,), dtype=jnp.int32)
    for k in range(_RPW):
        if k + 1 < _RPW:
            copies.append(
                pltpu.async_copy(
                    x_hbm.at[row0 + k + 1], bufs[(k + 1) % 2], sems[(k + 1) % 2]
                )
            )
        copies[k].wait()
        r = _row_argmax(bufs[k % 2])
        acc = jnp.where(iota == k, r, acc)
    outv[...] = acc
    pltpu.sync_copy(outv, out_hbm.at[wid])


def kernel(x):
    out = _argmax_sc(x)
    return out[:, :_RPW].reshape(_ROWS)


# speedup vs baseline: 1.0247x; 1.0247x over previous
import functools

import jax
import jax.numpy as jnp
from jax import lax
from jax.experimental import pallas as pl
from jax.experimental.pallas import tpu as pltpu
from jax.experimental.pallas import tpu_sc as plsc

_ROWS = 128
_COLS = 32768
_NC = 2
_NS = 16
_NW = _NC * _NS
_RPW = _ROWS // _NW
_L = 16
_UNROLL = 8
_NIT = _COLS // (_L * _UNROLL)


def _permute(x, perm):
    dn = lax.GatherDimensionNumbers(
        offset_dims=(), collapsed_slice_dims=(0,), start_index_map=(0,)
    )
    return lax.gather(
        x, perm[:, None], dn, slice_sizes=(1,),
        mode=lax.GatherScatterMode.PROMISE_IN_BOUNDS,
    )


def _merge(m, e, pm, pe):
    better = (pm > m) | ((pm == m) & (pe < e))
    return jnp.where(better, pm, m), jnp.where(better, pe, e)


def _row_argmax(buf):
    iota = lax.iota(jnp.int32, _L)
    neg = jnp.full((_L,), -jnp.inf, dtype=jnp.float32)
    zero = jnp.zeros((_L,), dtype=jnp.int32)

    def step(it, carry):
        ms = carry[:_UNROLL]
        js = carry[_UNROLL:]
        new_ms, new_js = [], []
        base = it * _UNROLL
        for a in range(_UNROLL):
            j = base + a
            v = buf[pl.ds(j * _L, _L)]
            gt = v > ms[a]
            new_ms.append(jnp.where(gt, v, ms[a]))
            new_js.append(jnp.where(gt, j, js[a]))
        return tuple(new_ms) + tuple(new_js)

    carry = lax.fori_loop(0, _NIT, step, (neg,) * _UNROLL + (zero,) * _UNROLL)
    ms = carry[:_UNROLL]
    js = carry[_UNROLL:]

    m, e = ms[0], js[0] * _L + iota
    for a in range(1, _UNROLL):
        m, e = _merge(m, e, ms[a], js[a] * _L + iota)

    km = m
    for s in (8, 4, 2, 1):
        km = jnp.maximum(km, _permute(km, iota ^ s))
    cand = jnp.where(m == km, e, jnp.int32(_COLS))
    for s in (8, 4, 2, 1):
        cand = jnp.minimum(cand, _permute(cand, iota ^ s))
    return cand


@functools.partial(
    pl.kernel,
    out_type=jax.ShapeDtypeStruct((_NW, _L), jnp.int32),
    mesh=plsc.VectorSubcoreMesh(core_axis_name="c", subcore_axis_name="s"),
    scratch_types=[
        pltpu.VMEM((_COLS,), jnp.float32),
        pltpu.VMEM((_COLS,), jnp.float32),
        pltpu.VMEM((_L,), jnp.int32),
        pltpu.SemaphoreType.DMA,
        pltpu.SemaphoreType.DMA,
    ],
)
def _argmax_sc(x_hbm, out_hbm, buf0, buf1, outv, sem0, sem1):
    wid = lax.axis_index("c") * _NS + lax.axis_index("s")
    row0 = wid * _RPW
    iota = lax.iota(jnp.int32, _L)

    pltpu.async_copy(x_hbm.at[row0], buf0, sem0)
    pltpu.async_copy(x_hbm.at[row0 + 1], buf1, sem1)

    def pair(k2, acc):
        r = row0 + 2 * k2
        pltpu.make_async_copy(x_hbm.at[r], buf0, sem0).wait()
        bi0 = _row_argmax(buf0)
        acc = jnp.where(iota == 2 * k2, bi0, acc)

        @pl.when(k2 + 1 < _RPW // 2)
        def _():
            pltpu.async_copy(x_hbm.at[r + 2], buf0, sem0)

        pltpu.make_async_copy(x_hbm.at[r], buf1, sem1).wait()
        bi1 = _row_argmax(buf1)
        acc = jnp.where(iota == 2 * k2 + 1, bi1, acc)

        @pl.when(k2 + 1 < _RPW // 2)
        def _():
            pltpu.async_copy(x_hbm.at[r + 3], buf1, sem1)

        return acc

    acc = lax.fori_loop(0, _RPW // 2, pair, jnp.zeros((_L,), jnp.int32))
    outv[...] = acc
    pltpu.sync_copy(outv, out_hbm.at[wid])


def kernel(x):
    out = _argmax_sc(x)
    return out[:, :_RPW].reshape(_ROWS)
